# Initial kernel scaffold; baseline (speedup 1.0000x reference)
import jax
import jax.numpy as jnp
from jax.experimental import pallas as pl


def kernel(x, edge_index, W1_l, W1_r, b1, W2_l, W2_r, b2, lin_W, lin_b):
    raise NotImplementedError("write your pallas kernel here")


# speedup vs baseline: 5.4577x; 5.4577x over previous
import jax
import jax.numpy as jnp
from jax import lax
from jax.experimental import pallas as pl
from jax.experimental.pallas import tpu as pltpu
from jax.experimental.pallas import tpu_sc as plsc

N = 10000
E = 320000
D = 128
H = 128
C = 16
MAXNORM = 1.0 - 4e-3

NC = 2
NS = 16
NW = NC * NS
EPW = E // NW
K = 80
CHUNKS = EPW // K
NP_ = 10240
RPT = NP_ // NS
CW = 16


_SC_AGG_CACHE = {}


def _make_sc_agg(width, with_counts):
    key = (width, with_counts)
    if key in _SC_AGG_CACHE:
        return _SC_AGG_CACHE[key]
    mesh = plsc.VectorSubcoreMesh(core_axis_name="c", subcore_axis_name="s",
                                  num_cores=NC, num_subcores=NS)
    outs = [jax.ShapeDtypeStruct((NC * NP_, width), jnp.float32)]
    scratch = [
        pltpu.VMEM((K,), jnp.int32),
        pltpu.VMEM((K,), jnp.int32),
        pltpu.VMEM((K, width), jnp.float32),
        pltpu.VMEM_SHARED((NP_, width), jnp.float32),
        pltpu.SemaphoreType.DMA,
    ]
    if with_counts:
        outs.append(jax.ShapeDtypeStruct((NC * NP_, CW), jnp.float32))
        scratch += [
            pltpu.VMEM((K, CW), jnp.float32),
            pltpu.VMEM_SHARED((NP_, CW), jnp.float32),
        ]

    def body(*refs):
        if with_counts:
            (table, srci, dsti, zrow, zc, ones_h,
             out, cnt_out, sidx, didx, rows, acc_sh, sem, ones_v, cnt_sh) = refs
        else:
            (table, srci, dsti, zrow,
             out, sidx, didx, rows, acc_sh, sem) = refs
        cid = lax.axis_index("c")
        sid = lax.axis_index("s")
        wid = sid * NC + cid
        r0 = sid * RPT
        pltpu.sync_copy(zrow, acc_sh.at[pl.ds(r0, RPT)])
        if with_counts:
            pltpu.sync_copy(zc, cnt_sh.at[pl.ds(r0, RPT)])
            pltpu.sync_copy(ones_h, ones_v)
        plsc.subcore_barrier()

        e0 = wid * EPW

        def step(i, carry):
            base = e0 + i * K
            pltpu.sync_copy(srci.at[pl.ds(base, K)], sidx)
            pltpu.sync_copy(dsti.at[pl.ds(base, K)], didx)
            pltpu.async_copy(table.at[sidx], rows, sem).wait()
            pltpu.sync_copy(rows, acc_sh.at[didx], add=True)
            if with_counts:
                pltpu.sync_copy(ones_v, cnt_sh.at[didx], add=True)
            return carry

        lax.fori_loop(0, CHUNKS, step, 0)
        plsc.subcore_barrier()
        o0 = cid * NP_ + r0
        pltpu.sync_copy(acc_sh.at[pl.ds(r0, RPT)], out.at[pl.ds(o0, RPT)])
        if with_counts:
            pltpu.sync_copy(cnt_sh.at[pl.ds(r0, RPT)], cnt_out.at[pl.ds(o0, RPT)])

    k = pl.kernel(body, out_type=tuple(outs), mesh=mesh,
                  scratch_types=scratch,
                  compiler_params=pltpu.CompilerParams(
                      use_tc_tiling_on_sc=False))
    _SC_AGG_CACHE[key] = k
    return k


BN = 400
GRID = N // BN


def _tcb_body(a0, a1, cntT, x, w1l, w1r, b1, w2l, w2r, b2, g_out, hr_out):
    cnt = jnp.maximum(jnp.sum(cntT[...], axis=1, keepdims=True), 1.0)
    mean = (a0[...] + a1[...]) / cnt
    h = jnp.dot(mean, w1l[...], preferred_element_type=jnp.float32)
    h += jnp.dot(x[...], w1r[...], preferred_element_type=jnp.float32)
    h = jnp.maximum(h + b1[...], 0.0)
    g_out[...] = jnp.dot(h, w2l[...], preferred_element_type=jnp.float32)
    hr_out[...] = jnp.dot(h, w2r[...], preferred_element_type=jnp.float32) + b2[...]


_tcb = pl.pallas_call(
    _tcb_body,
    grid=(GRID,),
    in_specs=[
        pl.BlockSpec((BN, D), lambda i: (i, 0)),
        pl.BlockSpec((BN, D), lambda i: (i, 0)),
        pl.BlockSpec((BN, 2), lambda i: (i, 0)),
        pl.BlockSpec((BN, D), lambda i: (i, 0)),
        pl.BlockSpec((D, H), lambda i: (0, 0)),
        pl.BlockSpec((D, H), lambda i: (0, 0)),
        pl.BlockSpec((1, H), lambda i: (0, 0)),
        pl.BlockSpec((H, C), lambda i: (0, 0)),
        pl.BlockSpec((H, C), lambda i: (0, 0)),
        pl.BlockSpec((1, C), lambda i: (0, 0)),
    ],
    out_specs=[
        pl.BlockSpec((BN, C), lambda i: (i, 0)),
        pl.BlockSpec((BN, C), lambda i: (i, 0)),
    ],
    out_shape=[
        jax.ShapeDtypeStruct((N, C), jnp.float32),
        jax.ShapeDtypeStruct((N, C), jnp.float32),
    ],
)


def _tcc_body(a0, a1, cntT, hr, lw, lb, out):
    cnt = jnp.maximum(jnp.sum(cntT[...], axis=1, keepdims=True), 1.0)
    h2 = (a0[...] + a1[...]) / cnt + hr[...]
    nsq = jnp.sum(h2 * h2, axis=1, keepdims=True)
    norm_o = jnp.maximum(nsq, 1e-15)
    s1 = jnp.where(norm_o > MAXNORM, MAXNORM / norm_o, 1.0)
    p_norm = jnp.maximum(s1 * norm_o, 1e-15)
    z = jnp.clip(p_norm, -1.0 + 1e-7, 1.0 - 1e-7)
    artanh = 0.5 * jnp.log((1.0 + z) / (1.0 - z))
    alpha = s1 * artanh / p_norm
    t = jnp.dot(h2, lw[...], preferred_element_type=jnp.float32)
    acc = h2[:, 0:1] * t[:, 0:C]
    for j in range(1, C):
        acc += h2[:, j:j + 1] * t[:, j * C:(j + 1) * C]
    h_euc = alpha * acc + lb[...]
    u_norm = jnp.maximum(
        jnp.sqrt(jnp.sum(h_euc * h_euc, axis=1, keepdims=True)), 1e-15)
    gamma = jnp.tanh(u_norm) * h_euc / u_norm
    n2 = jnp.maximum(
        jnp.sqrt(jnp.sum(gamma * gamma, axis=1, keepdims=True)), 1e-15)
    gamma = jnp.where(n2 > MAXNORM, gamma * (MAXNORM / n2), gamma)
    m = jnp.max(gamma, axis=1, keepdims=True)
    y = gamma - m
    out[...] = y - jnp.log(jnp.sum(jnp.exp(y), axis=1, keepdims=True))


_tcc = pl.pallas_call(
    _tcc_body,
    grid=(GRID,),
    in_specs=[
        pl.BlockSpec((BN, C), lambda i: (i, 0)),
        pl.BlockSpec((BN, C), lambda i: (i, 0)),
        pl.BlockSpec((BN, 2), lambda i: (i, 0)),
        pl.BlockSpec((BN, C), lambda i: (i, 0)),
        pl.BlockSpec((C, C * C), lambda i: (0, 0)),
        pl.BlockSpec((1, C), lambda i: (0, 0)),
    ],
    out_specs=pl.BlockSpec((BN, C), lambda i: (i, 0)),
    out_shape=jax.ShapeDtypeStruct((N, C), jnp.float32),
)


def kernel(x, edge_index, W1_l, W1_r, b1, W2_l, W2_r, b2, lin_W, lin_b):
    src = edge_index[0].astype(jnp.int32)
    dst = edge_index[1].astype(jnp.int32)
    zrow_d = jnp.zeros((RPT, D), jnp.float32)
    zrow_c = jnp.zeros((RPT, C), jnp.float32)
    zc = jnp.zeros((RPT, CW), jnp.float32)
    ones_h = jnp.ones((K, CW), jnp.float32)

    acc1, cnt = _make_sc_agg(D, True)(x, src, dst, zrow_d, zc, ones_h)
    cntT = jnp.concatenate([cnt[:N, 0:1], cnt[NP_:NP_ + N, 0:1]], axis=1)
    g, hr = _tcb(acc1[:N], acc1[NP_:NP_ + N], cntT, x, W1_l, W1_r,
                 b1.reshape(1, H), W2_l, W2_r, b2.reshape(1, C))
    (acc2,) = _make_sc_agg(C, False)(g, src, dst, zrow_c)
    out = _tcc(acc2[:N], acc2[NP_:NP_ + N], cntT, hr,
               lin_W.reshape(C, C * C), lin_b.reshape(1, C))
    return out


# speedup vs baseline: 7.7944x; 1.4281x over previous
import jax
import jax.numpy as jnp
from jax import lax
from jax.experimental import pallas as pl
from jax.experimental.pallas import tpu as pltpu
from jax.experimental.pallas import tpu_sc as plsc

N = 10000
E = 320000
D = 128
H = 128
C = 16
MAXNORM = 1.0 - 4e-3

NC = 2
NS = 16
NW = NC * NS
EPW = E // NW
K = 80
CHUNKS = EPW // K
NP_ = 10240
RPT = NP_ // NS
CW = 16


_SC_AGG_CACHE = {}


def _make_sc_agg(width, with_counts):
    key = (width, with_counts)
    if key in _SC_AGG_CACHE:
        return _SC_AGG_CACHE[key]
    mesh = plsc.VectorSubcoreMesh(core_axis_name="c", subcore_axis_name="s",
                                  num_cores=NC, num_subcores=NS)
    outs = [jax.ShapeDtypeStruct((NC * NP_, width), jnp.float32)]
    scratch = [
        pltpu.VMEM((K,), jnp.int32),
        pltpu.VMEM((K,), jnp.int32),
        pltpu.VMEM((K,), jnp.int32),
        pltpu.VMEM((K,), jnp.int32),
        pltpu.VMEM((K, width), jnp.float32),
        pltpu.VMEM((K, width), jnp.float32),
        pltpu.VMEM_SHARED((NP_, width), jnp.float32),
        pltpu.SemaphoreType.DMA,
        pltpu.SemaphoreType.DMA,
    ]
    if with_counts:
        outs.append(jax.ShapeDtypeStruct((NC * NP_, CW), jnp.float32))
        scratch += [
            pltpu.VMEM((K, CW), jnp.float32),
            pltpu.VMEM_SHARED((NP_, CW), jnp.float32),
        ]

    def body(*refs):
        if with_counts:
            (table, srci, dsti, zrow, zc, ones_h, out, cnt_out,
             sidx0, sidx1, didx0, didx1, rows0, rows1, acc_sh,
             sem0, sem1, ones_v, cnt_sh) = refs
        else:
            (table, srci, dsti, zrow, out,
             sidx0, sidx1, didx0, didx1, rows0, rows1, acc_sh,
             sem0, sem1) = refs
        sidx = (sidx0, sidx1)
        didx = (didx0, didx1)
        rows = (rows0, rows1)
        sem = (sem0, sem1)
        cid = lax.axis_index("c")
        sid = lax.axis_index("s")
        wid = sid * NC + cid
        r0 = sid * RPT
        pltpu.sync_copy(zrow, acc_sh.at[pl.ds(r0, RPT)])
        if with_counts:
            pltpu.sync_copy(zc, cnt_sh.at[pl.ds(r0, RPT)])
            pltpu.sync_copy(ones_h, ones_v)
        plsc.subcore_barrier()

        e0 = wid * EPW

        def fetch(s, i):
            base = e0 + i * K
            pltpu.sync_copy(srci.at[pl.ds(base, K)], sidx[s])
            pltpu.sync_copy(dsti.at[pl.ds(base, K)], didx[s])
            pltpu.async_copy(table.at[sidx[s]], rows[s], sem[s])

        def drain(s):
            pltpu.make_async_copy(table.at[sidx[s]], rows[s], sem[s]).wait()

        def consume(s):
            drain(s)
            pltpu.sync_copy(rows[s], acc_sh.at[didx[s]], add=True)
            if with_counts:
                pltpu.sync_copy(ones_v, cnt_sh.at[didx[s]], add=True)

        fetch(0, 0)
        fetch(1, 1)

        def step(p, carry):
            i = p * 2
            consume(0)
            fetch(0, i + 2)
            consume(1)

            @pl.when(i + 3 < CHUNKS)
            def _():
                fetch(1, i + 3)

            return carry

        lax.fori_loop(0, (CHUNKS - 1) // 2, step, 0)
        consume(0)
        plsc.subcore_barrier()
        o0 = cid * NP_ + r0
        pltpu.sync_copy(acc_sh.at[pl.ds(r0, RPT)], out.at[pl.ds(o0, RPT)])
        if with_counts:
            pltpu.sync_copy(cnt_sh.at[pl.ds(r0, RPT)], cnt_out.at[pl.ds(o0, RPT)])

    k = pl.kernel(body, out_type=tuple(outs), mesh=mesh,
                  scratch_types=scratch,
                  compiler_params=pltpu.CompilerParams(
                      use_tc_tiling_on_sc=False))
    _SC_AGG_CACHE[key] = k
    return k


BN = 400
GRID = N // BN


def _tcb_body(a0, a1, cntT, x, w1l, w1r, b1, w2l, w2r, b2, g_out, hr_out):
    cnt = jnp.maximum(jnp.sum(cntT[...], axis=1, keepdims=True), 1.0)
    mean = (a0[...] + a1[...]) / cnt
    h = jnp.dot(mean, w1l[...], preferred_element_type=jnp.float32)
    h += jnp.dot(x[...], w1r[...], preferred_element_type=jnp.float32)
    h = jnp.maximum(h + b1[...], 0.0)
    g_out[...] = jnp.dot(h, w2l[...], preferred_element_type=jnp.float32)
    hr_out[...] = jnp.dot(h, w2r[...], preferred_element_type=jnp.float32) + b2[...]


_tcb = pl.pallas_call(
    _tcb_body,
    grid=(GRID,),
    in_specs=[
        pl.BlockSpec((BN, D), lambda i: (i, 0)),
        pl.BlockSpec((BN, D), lambda i: (i, 0)),
        pl.BlockSpec((BN, 2), lambda i: (i, 0)),
        pl.BlockSpec((BN, D), lambda i: (i, 0)),
        pl.BlockSpec((D, H), lambda i: (0, 0)),
        pl.BlockSpec((D, H), lambda i: (0, 0)),
        pl.BlockSpec((1, H), lambda i: (0, 0)),
        pl.BlockSpec((H, C), lambda i: (0, 0)),
        pl.BlockSpec((H, C), lambda i: (0, 0)),
        pl.BlockSpec((1, C), lambda i: (0, 0)),
    ],
    out_specs=[
        pl.BlockSpec((BN, C), lambda i: (i, 0)),
        pl.BlockSpec((BN, C), lambda i: (i, 0)),
    ],
    out_shape=[
        jax.ShapeDtypeStruct((N, C), jnp.float32),
        jax.ShapeDtypeStruct((N, C), jnp.float32),
    ],
)


def _tcc_body(a0, a1, cntT, hr, lw, lb, out):
    cnt = jnp.maximum(jnp.sum(cntT[...], axis=1, keepdims=True), 1.0)
    h2 = (a0[...] + a1[...]) / cnt + hr[...]
    nsq = jnp.sum(h2 * h2, axis=1, keepdims=True)
    norm_o = jnp.maximum(nsq, 1e-15)
    s1 = jnp.where(norm_o > MAXNORM, MAXNORM / norm_o, 1.0)
    p_norm = jnp.maximum(s1 * norm_o, 1e-15)
    z = jnp.clip(p_norm, -1.0 + 1e-7, 1.0 - 1e-7)
    artanh = 0.5 * jnp.log((1.0 + z) / (1.0 - z))
    alpha = s1 * artanh / p_norm
    t = jnp.dot(h2, lw[...], preferred_element_type=jnp.float32)
    acc = h2[:, 0:1] * t[:, 0:C]
    for j in range(1, C):
        acc += h2[:, j:j + 1] * t[:, j * C:(j + 1) * C]
    h_euc = alpha * acc + lb[...]
    u_norm = jnp.maximum(
        jnp.sqrt(jnp.sum(h_euc * h_euc, axis=1, keepdims=True)), 1e-15)
    gamma = jnp.tanh(u_norm) * h_euc / u_norm
    n2 = jnp.maximum(
        jnp.sqrt(jnp.sum(gamma * gamma, axis=1, keepdims=True)), 1e-15)
    gamma = jnp.where(n2 > MAXNORM, gamma * (MAXNORM / n2), gamma)
    m = jnp.max(gamma, axis=1, keepdims=True)
    y = gamma - m
    out[...] = y - jnp.log(jnp.sum(jnp.exp(y), axis=1, keepdims=True))


_tcc = pl.pallas_call(
    _tcc_body,
    grid=(GRID,),
    in_specs=[
        pl.BlockSpec((BN, C), lambda i: (i, 0)),
        pl.BlockSpec((BN, C), lambda i: (i, 0)),
        pl.BlockSpec((BN, 2), lambda i: (i, 0)),
        pl.BlockSpec((BN, C), lambda i: (i, 0)),
        pl.BlockSpec((C, C * C), lambda i: (0, 0)),
        pl.BlockSpec((1, C), lambda i: (0, 0)),
    ],
    out_specs=pl.BlockSpec((BN, C), lambda i: (i, 0)),
    out_shape=jax.ShapeDtypeStruct((N, C), jnp.float32),
)


def kernel(x, edge_index, W1_l, W1_r, b1, W2_l, W2_r, b2, lin_W, lin_b):
    src = edge_index[0].astype(jnp.int32)
    dst = edge_index[1].astype(jnp.int32)
    zrow_d = jnp.zeros((RPT, D), jnp.float32)
    zrow_c = jnp.zeros((RPT, C), jnp.float32)
    zc = jnp.zeros((RPT, CW), jnp.float32)
    ones_h = jnp.ones((K, CW), jnp.float32)

    acc1, cnt = _make_sc_agg(D, True)(x, src, dst, zrow_d, zc, ones_h)
    cntT = jnp.concatenate([cnt[:N, 0:1], cnt[NP_:NP_ + N, 0:1]], axis=1)
    g, hr = _tcb(acc1[:N], acc1[NP_:NP_ + N], cntT, x, W1_l, W1_r,
                 b1.reshape(1, H), W2_l, W2_r, b2.reshape(1, C))
    (acc2,) = _make_sc_agg(C, False)(g, src, dst, zrow_c)
    out = _tcc(acc2[:N], acc2[NP_:NP_ + N], cntT, hr,
               lin_W.reshape(C, C * C), lin_b.reshape(1, C))
    return out


# speedup vs baseline: 8.7539x; 1.1231x over previous
import jax
import jax.numpy as jnp
from jax import lax
from jax.experimental import pallas as pl
from jax.experimental.pallas import tpu as pltpu
from jax.experimental.pallas import tpu_sc as plsc

N = 10000
E = 320000
D = 128
H = 128
C = 16
MAXNORM = 1.0 - 4e-3

NC = 2
NS = 16
NW = NC * NS
EPW = E // NW
K = 80
CHUNKS = EPW // K
NP_ = 10240
RPT = NP_ // NS
CW = 16


_SC_AGG_CACHE = {}


def _make_sc_agg(width, with_counts):
    key = (width, with_counts)
    if key in _SC_AGG_CACHE:
        return _SC_AGG_CACHE[key]
    mesh = plsc.VectorSubcoreMesh(core_axis_name="c", subcore_axis_name="s",
                                  num_cores=NC, num_subcores=NS)
    outs = [jax.ShapeDtypeStruct((NC * NP_, width), jnp.float32)]
    scratch = [
        pltpu.VMEM((K,), jnp.int32),
        pltpu.VMEM((K,), jnp.int32),
        pltpu.VMEM((K,), jnp.int32),
        pltpu.VMEM((K,), jnp.int32),
        pltpu.VMEM((K, width), jnp.float32),
        pltpu.VMEM((K, width), jnp.float32),
        pltpu.VMEM_SHARED((NP_, width), jnp.float32),
        pltpu.SemaphoreType.DMA,
        pltpu.SemaphoreType.DMA,
    ]
    if with_counts:
        outs.append(jax.ShapeDtypeStruct((NC * NP_, CW), jnp.float32))
        scratch += [
            pltpu.VMEM((K, CW), jnp.float32),
            pltpu.VMEM_SHARED((NP_, CW), jnp.float32),
        ]

    def body(*refs):
        if with_counts:
            (table, srci, dsti, zrow, zc, ones_h, out, cnt_out,
             sidx0, sidx1, didx0, didx1, rows0, rows1, acc_sh,
             sem0, sem1, ones_v, cnt_sh) = refs
        else:
            (table, srci, dsti, zrow, out,
             sidx0, sidx1, didx0, didx1, rows0, rows1, acc_sh,
             sem0, sem1) = refs
        sidx = (sidx0, sidx1)
        didx = (didx0, didx1)
        rows = (rows0, rows1)
        sem = (sem0, sem1)
        cid = lax.axis_index("c")
        sid = lax.axis_index("s")
        wid = sid * NC + cid
        r0 = sid * RPT
        pltpu.sync_copy(zrow, acc_sh.at[pl.ds(r0, RPT)])
        if with_counts:
            pltpu.sync_copy(zc, cnt_sh.at[pl.ds(r0, RPT)])
            pltpu.sync_copy(ones_h, ones_v)
        plsc.subcore_barrier()

        e0 = wid * EPW

        def fetch(s, i):
            base = e0 + i * K
            pltpu.sync_copy(srci.at[pl.ds(base, K)], sidx[s])
            pltpu.sync_copy(dsti.at[pl.ds(base, K)], didx[s])
            pltpu.async_copy(table.at[sidx[s]], rows[s], sem[s])

        def drain(s):
            pltpu.make_async_copy(table.at[sidx[s]], rows[s], sem[s]).wait()

        def consume(s):
            drain(s)
            pltpu.sync_copy(rows[s], acc_sh.at[didx[s]], add=True)
            if with_counts:
                pltpu.sync_copy(ones_v, cnt_sh.at[didx[s]], add=True)

        fetch(0, 0)
        fetch(1, 1)

        def step(p, carry):
            i = p * 2
            consume(0)
            fetch(0, i + 2)
            consume(1)

            @pl.when(i + 3 < CHUNKS)
            def _():
                fetch(1, i + 3)

            return carry

        lax.fori_loop(0, (CHUNKS - 1) // 2, step, 0)
        consume(0)
        plsc.subcore_barrier()
        o0 = cid * NP_ + r0
        pltpu.sync_copy(acc_sh.at[pl.ds(r0, RPT)], out.at[pl.ds(o0, RPT)])
        if with_counts:
            pltpu.sync_copy(cnt_sh.at[pl.ds(r0, RPT)], cnt_out.at[pl.ds(o0, RPT)])

    k = pl.kernel(body, out_type=tuple(outs), mesh=mesh,
                  scratch_types=scratch,
                  compiler_params=pltpu.CompilerParams(
                      use_tc_tiling_on_sc=False))
    _SC_AGG_CACHE[key] = k
    return k


BN = 1000
GRID = N // BN


def _tcb_body(a0, a1, cntT, x, w1l, w1r, b1, w2l, w2r, b2, g_out, hr_out):
    cnt = jnp.maximum(jnp.sum(cntT[...], axis=1, keepdims=True), 1.0)
    mean = (a0[...] + a1[...]) / cnt
    h = jnp.dot(mean, w1l[...], preferred_element_type=jnp.float32)
    h += jnp.dot(x[...], w1r[...], preferred_element_type=jnp.float32)
    h = jnp.maximum(h + b1[...], 0.0)
    g_out[...] = jnp.dot(h, w2l[...], preferred_element_type=jnp.float32)
    hr_out[...] = jnp.dot(h, w2r[...], preferred_element_type=jnp.float32) + b2[...]


_tcb = pl.pallas_call(
    _tcb_body,
    grid=(GRID,),
    in_specs=[
        pl.BlockSpec((BN, D), lambda i: (i, 0)),
        pl.BlockSpec((BN, D), lambda i: (i, 0)),
        pl.BlockSpec((BN, 2), lambda i: (i, 0)),
        pl.BlockSpec((BN, D), lambda i: (i, 0)),
        pl.BlockSpec((D, H), lambda i: (0, 0)),
        pl.BlockSpec((D, H), lambda i: (0, 0)),
        pl.BlockSpec((1, H), lambda i: (0, 0)),
        pl.BlockSpec((H, C), lambda i: (0, 0)),
        pl.BlockSpec((H, C), lambda i: (0, 0)),
        pl.BlockSpec((1, C), lambda i: (0, 0)),
    ],
    out_specs=[
        pl.BlockSpec((BN, C), lambda i: (i, 0)),
        pl.BlockSpec((BN, C), lambda i: (i, 0)),
    ],
    out_shape=[
        jax.ShapeDtypeStruct((N, C), jnp.float32),
        jax.ShapeDtypeStruct((N, C), jnp.float32),
    ],
)


def _tcc_body(a0, a1, cntT, hr, ra, rb, lw, lb, out):
    cnt = jnp.maximum(jnp.sum(cntT[...], axis=1, keepdims=True), 1.0)
    h2 = (a0[...] + a1[...]) / cnt + hr[...]
    nsq = jnp.sum(h2 * h2, axis=1, keepdims=True)
    norm_o = jnp.maximum(nsq, 1e-15)
    s1 = jnp.where(norm_o > MAXNORM, MAXNORM / norm_o, 1.0)
    p_norm = jnp.maximum(s1 * norm_o, 1e-15)
    z = jnp.clip(p_norm, -1.0 + 1e-7, 1.0 - 1e-7)
    artanh = 0.5 * jnp.log((1.0 + z) / (1.0 - z))
    alpha = s1 * artanh / p_norm
    a = jnp.dot(h2, ra[...], preferred_element_type=jnp.float32)
    b = jnp.dot(h2, rb[...], preferred_element_type=jnp.float32)
    acc = jnp.dot(a * b, lw[...], preferred_element_type=jnp.float32)
    h_euc = alpha * acc + lb[...]
    u_norm = jnp.maximum(
        jnp.sqrt(jnp.sum(h_euc * h_euc, axis=1, keepdims=True)), 1e-15)
    gamma = jnp.tanh(u_norm) * h_euc / u_norm
    n2 = jnp.maximum(
        jnp.sqrt(jnp.sum(gamma * gamma, axis=1, keepdims=True)), 1e-15)
    gamma = jnp.where(n2 > MAXNORM, gamma * (MAXNORM / n2), gamma)
    m = jnp.max(gamma, axis=1, keepdims=True)
    y = gamma - m
    out[...] = y - jnp.log(jnp.sum(jnp.exp(y), axis=1, keepdims=True))


_tcc = pl.pallas_call(
    _tcc_body,
    grid=(GRID,),
    in_specs=[
        pl.BlockSpec((BN, C), lambda i: (i, 0)),
        pl.BlockSpec((BN, C), lambda i: (i, 0)),
        pl.BlockSpec((BN, 2), lambda i: (i, 0)),
        pl.BlockSpec((BN, C), lambda i: (i, 0)),
        pl.BlockSpec((C, C * C), lambda i: (0, 0)),
        pl.BlockSpec((C, C * C), lambda i: (0, 0)),
        pl.BlockSpec((C * C, C), lambda i: (0, 0)),
        pl.BlockSpec((1, C), lambda i: (0, 0)),
    ],
    out_specs=pl.BlockSpec((BN, C), lambda i: (i, 0)),
    out_shape=jax.ShapeDtypeStruct((N, C), jnp.float32),
)


def kernel(x, edge_index, W1_l, W1_r, b1, W2_l, W2_r, b2, lin_W, lin_b):
    src = edge_index[0].astype(jnp.int32)
    dst = edge_index[1].astype(jnp.int32)
    zrow_d = jnp.zeros((RPT, D), jnp.float32)
    zrow_c = jnp.zeros((RPT, C), jnp.float32)
    zc = jnp.zeros((RPT, CW), jnp.float32)
    ones_h = jnp.ones((K, CW), jnp.float32)

    acc1, cnt = _make_sc_agg(D, True)(x, src, dst, zrow_d, zc, ones_h)
    cntT = jnp.concatenate([cnt[:N, 0:1], cnt[NP_:NP_ + N, 0:1]], axis=1)
    g, hr = _tcb(acc1[:N], acc1[NP_:NP_ + N], cntT, x, W1_l, W1_r,
                 b1.reshape(1, H), W2_l, W2_r, b2.reshape(1, C))
    (acc2,) = _make_sc_agg(C, False)(g, src, dst, zrow_c)
    ra = jnp.repeat(jnp.eye(C, dtype=jnp.float32), C, axis=1)
    rb = jnp.tile(jnp.eye(C, dtype=jnp.float32), (1, C))
    out = _tcc(acc2[:N], acc2[NP_:NP_ + N], cntT, hr,
               ra, rb, lin_W, lin_b.reshape(1, C))
    return out


# speedup vs baseline: 11.3203x; 1.2932x over previous
import jax
import jax.numpy as jnp
from jax import lax
from jax.experimental import pallas as pl
from jax.experimental.pallas import tpu as pltpu
from jax.experimental.pallas import tpu_sc as plsc

N = 10000
E = 320000
D = 128
H = 128
C = 16
MAXNORM = 1.0 - 4e-3

NC = 2
NS = 16
NW = NC * NS
EPW = E // NW
K = 80
CHUNKS = EPW // K
NP_ = 10240
RPT = NP_ // NS
CW = 16
NSLOT = 5


_SC_AGG_CACHE = {}


def _make_sc_agg(width, with_counts):
    key = (width, with_counts)
    if key in _SC_AGG_CACHE:
        return _SC_AGG_CACHE[key]
    mesh = plsc.VectorSubcoreMesh(core_axis_name="c", subcore_axis_name="s",
                                  num_cores=NC, num_subcores=NS)
    nslot = 3 if width >= 128 else 6
    outs = [jax.ShapeDtypeStruct((NC * NP_, width), jnp.float32)]
    scratch = [
        pltpu.VMEM((nslot, 2, K), jnp.int32),
        pltpu.VMEM((nslot, K, width), jnp.float32),
        pltpu.VMEM_SHARED((NP_, width), jnp.float32),
        pltpu.SemaphoreType.DMA((nslot,)),
        pltpu.SemaphoreType.DMA((nslot,)),
        pltpu.SemaphoreType.DMA((nslot,)),
    ]
    if with_counts:
        outs.append(jax.ShapeDtypeStruct((NC * NP_, CW), jnp.float32))
        scratch += [
            pltpu.VMEM((K, CW), jnp.float32),
            pltpu.VMEM_SHARED((NP_, CW), jnp.float32),
        ]

    def body(*refs):
        if with_counts:
            (table, eidx3, zrow, zc, ones_h, out, cnt_out,
             eidx, rows, acc_sh, isem, gsem, ssem, ones_v, cnt_sh) = refs
        else:
            (table, eidx3, zrow, out,
             eidx, rows, acc_sh, isem, gsem, ssem) = refs
        cid = lax.axis_index("c")
        sid = lax.axis_index("s")
        wid = sid * NC + cid
        r0 = sid * RPT
        c0 = wid * CHUNKS
        pltpu.sync_copy(zrow, acc_sh.at[pl.ds(r0, RPT)])
        if with_counts:
            pltpu.sync_copy(zc, cnt_sh.at[pl.ds(r0, RPT)])
            pltpu.sync_copy(ones_h, ones_v)
        plsc.subcore_barrier()

        def ifetch(s, i):
            pltpu.async_copy(eidx3.at[c0 + i], eidx.at[s], isem.at[s])

        def iwait(s, i):
            pltpu.make_async_copy(eidx3.at[c0 + i], eidx.at[s],
                                  isem.at[s]).wait()

        def gfetch(s):
            pltpu.async_copy(table.at[eidx.at[s, 0]], rows.at[s], gsem.at[s])

        def gwait(s):
            pltpu.make_async_copy(table.at[eidx.at[s, 0]], rows.at[s],
                                  gsem.at[s]).wait()

        def sissue(s):
            pltpu.async_copy(rows.at[s], acc_sh.at[eidx.at[s, 1]], ssem.at[s],
                             add=True)
            if with_counts:
                pltpu.async_copy(ones_v, cnt_sh.at[eidx.at[s, 1]], ssem.at[s],
                                 add=True)

        def swait(s):
            pltpu.make_async_copy(rows.at[s], acc_sh.at[eidx.at[s, 1]],
                                  ssem.at[s]).wait()
            if with_counts:
                pltpu.make_async_copy(ones_v, cnt_sh.at[eidx.at[s, 1]],
                                      ssem.at[s]).wait()

        for s in range(nslot):
            ifetch(s, s)
        for s in range(nslot):
            iwait(s, s)
            gfetch(s)

        ngroups = -(-CHUNKS // nslot)

        def step(p, carry):
            i0 = p * nslot
            for s in range(nslot):
                @pl.when(i0 + s < CHUNKS)
                def _(s=s):
                    gwait(s)
                    sissue(s)
            for s in range(nslot):
                j = i0 + nslot + s

                @pl.when(i0 + s < CHUNKS)
                def _(s=s, j=j):
                    swait(s)

                    @pl.when(j < CHUNKS)
                    def _(s=s, j=j):
                        ifetch(s, j)
                        iwait(s, j)
                        gfetch(s)

            return carry

        lax.fori_loop(0, ngroups, step, 0)
        plsc.subcore_barrier()
        o0 = cid * NP_ + r0
        pltpu.sync_copy(acc_sh.at[pl.ds(r0, RPT)], out.at[pl.ds(o0, RPT)])
        if with_counts:
            pltpu.sync_copy(cnt_sh.at[pl.ds(r0, RPT)], cnt_out.at[pl.ds(o0, RPT)])

    k = pl.kernel(body, out_type=tuple(outs), mesh=mesh,
                  scratch_types=scratch,
                  compiler_params=pltpu.CompilerParams(
                      use_tc_tiling_on_sc=False))
    _SC_AGG_CACHE[key] = k
    return k


BN = 1000
GRID = N // BN


def _tcb_body(a0, a1, cntT, x, w1l, w1r, b1, w2l, w2r, b2, g_out, hr_out):
    cnt = jnp.maximum(jnp.sum(cntT[...], axis=1, keepdims=True), 1.0)
    mean = (a0[...] + a1[...]) / cnt
    h = jnp.dot(mean, w1l[...], preferred_element_type=jnp.float32)
    h += jnp.dot(x[...], w1r[...], preferred_element_type=jnp.float32)
    h = jnp.maximum(h + b1[...], 0.0)
    g_out[...] = jnp.dot(h, w2l[...], preferred_element_type=jnp.float32)
    hr_out[...] = jnp.dot(h, w2r[...], preferred_element_type=jnp.float32) + b2[...]


_tcb = pl.pallas_call(
    _tcb_body,
    grid=(GRID,),
    in_specs=[
        pl.BlockSpec((BN, D), lambda i: (i, 0)),
        pl.BlockSpec((BN, D), lambda i: (i, 0)),
        pl.BlockSpec((BN, 2), lambda i: (i, 0)),
        pl.BlockSpec((BN, D), lambda i: (i, 0)),
        pl.BlockSpec((D, H), lambda i: (0, 0)),
        pl.BlockSpec((D, H), lambda i: (0, 0)),
        pl.BlockSpec((1, H), lambda i: (0, 0)),
        pl.BlockSpec((H, C), lambda i: (0, 0)),
        pl.BlockSpec((H, C), lambda i: (0, 0)),
        pl.BlockSpec((1, C), lambda i: (0, 0)),
    ],
    out_specs=[
        pl.BlockSpec((BN, C), lambda i: (i, 0)),
        pl.BlockSpec((BN, C), lambda i: (i, 0)),
    ],
    out_shape=[
        jax.ShapeDtypeStruct((N, C), jnp.float32),
        jax.ShapeDtypeStruct((N, C), jnp.float32),
    ],
)


def _tcc_body(a0, a1, cntT, hr, ra, rb, lw, lb, out):
    cnt = jnp.maximum(jnp.sum(cntT[...], axis=1, keepdims=True), 1.0)
    h2 = (a0[...] + a1[...]) / cnt + hr[...]
    nsq = jnp.sum(h2 * h2, axis=1, keepdims=True)
    norm_o = jnp.maximum(nsq, 1e-15)
    s1 = jnp.where(norm_o > MAXNORM, MAXNORM / norm_o, 1.0)
    p_norm = jnp.maximum(s1 * norm_o, 1e-15)
    z = jnp.clip(p_norm, -1.0 + 1e-7, 1.0 - 1e-7)
    artanh = 0.5 * jnp.log((1.0 + z) / (1.0 - z))
    alpha = s1 * artanh / p_norm
    a = jnp.dot(h2, ra[...], preferred_element_type=jnp.float32)
    b = jnp.dot(h2, rb[...], preferred_element_type=jnp.float32)
    acc = jnp.dot(a * b, lw[...], preferred_element_type=jnp.float32)
    h_euc = alpha * acc + lb[...]
    u_norm = jnp.maximum(
        jnp.sqrt(jnp.sum(h_euc * h_euc, axis=1, keepdims=True)), 1e-15)
    gamma = jnp.tanh(u_norm) * h_euc / u_norm
    n2 = jnp.maximum(
        jnp.sqrt(jnp.sum(gamma * gamma, axis=1, keepdims=True)), 1e-15)
    gamma = jnp.where(n2 > MAXNORM, gamma * (MAXNORM / n2), gamma)
    m = jnp.max(gamma, axis=1, keepdims=True)
    y = gamma - m
    out[...] = y - jnp.log(jnp.sum(jnp.exp(y), axis=1, keepdims=True))


_tcc = pl.pallas_call(
    _tcc_body,
    grid=(GRID,),
    in_specs=[
        pl.BlockSpec((BN, C), lambda i: (i, 0)),
        pl.BlockSpec((BN, C), lambda i: (i, 0)),
        pl.BlockSpec((BN, 2), lambda i: (i, 0)),
        pl.BlockSpec((BN, C), lambda i: (i, 0)),
        pl.BlockSpec((C, C * C), lambda i: (0, 0)),
        pl.BlockSpec((C, C * C), lambda i: (0, 0)),
        pl.BlockSpec((C * C, C), lambda i: (0, 0)),
        pl.BlockSpec((1, C), lambda i: (0, 0)),
    ],
    out_specs=pl.BlockSpec((BN, C), lambda i: (i, 0)),
    out_shape=jax.ShapeDtypeStruct((N, C), jnp.float32),
)


def kernel(x, edge_index, W1_l, W1_r, b1, W2_l, W2_r, b2, lin_W, lin_b):
    src = edge_index[0].astype(jnp.int32).reshape(NW, CHUNKS, 1, K)
    dst = edge_index[1].astype(jnp.int32).reshape(NW, CHUNKS, 1, K)
    eidx3 = jnp.concatenate([src, dst], axis=2).reshape(NW * CHUNKS, 2, K)
    zrow_d = jnp.zeros((RPT, D), jnp.float32)
    zrow_c = jnp.zeros((RPT, C), jnp.float32)
    zc = jnp.zeros((RPT, CW), jnp.float32)
    ones_h = jnp.ones((K, CW), jnp.float32)

    acc1, cnt = _make_sc_agg(D, True)(x, eidx3, zrow_d, zc, ones_h)
    cntT = jnp.concatenate([cnt[:N, 0:1], cnt[NP_:NP_ + N, 0:1]], axis=1)
    g, hr = _tcb(acc1[:N], acc1[NP_:NP_ + N], cntT, x, W1_l, W1_r,
                 b1.reshape(1, H), W2_l, W2_r, b2.reshape(1, C))
    (acc2,) = _make_sc_agg(C, False)(g, eidx3, zrow_c)
    ra = jnp.repeat(jnp.eye(C, dtype=jnp.float32), C, axis=1)
    rb = jnp.tile(jnp.eye(C, dtype=jnp.float32), (1, C))
    out = _tcc(acc2[:N], acc2[NP_:NP_ + N], cntT, hr,
               ra, rb, lin_W, lin_b.reshape(1, C))
    return out


# speedup vs baseline: 12.6908x; 1.1211x over previous
import jax
import jax.numpy as jnp
from jax import lax
from jax.experimental import pallas as pl
from jax.experimental.pallas import tpu as pltpu
from jax.experimental.pallas import tpu_sc as plsc

N = 10000
E = 320000
D = 128
H = 128
C = 16
MAXNORM = 1.0 - 4e-3

NC = 2
NS = 16
NW = NC * NS
EPW = E // NW
K = 128
CHUNKS = 79
PAD = CHUNKS * K - EPW
NP_ = 10240
RPT = NP_ // NS
CW = 16


_SC_AGG_CACHE = {}


def _make_sc_agg(width, with_counts):
    key = (width, with_counts)
    if key in _SC_AGG_CACHE:
        return _SC_AGG_CACHE[key]
    mesh = plsc.VectorSubcoreMesh(core_axis_name="c", subcore_axis_name="s",
                                  num_cores=NC, num_subcores=NS)
    nslot = 2 if width >= 128 else 4
    ngroups = -(-CHUNKS // nslot)
    assert ngroups % 2 == 0
    outs = [jax.ShapeDtypeStruct((NC * NP_, width), jnp.float32)]
    scratch = [
        pltpu.VMEM((2, nslot, 2, K), jnp.int32),
        pltpu.VMEM((nslot, K, width), jnp.float32),
        pltpu.VMEM_SHARED((NP_, width), jnp.float32),
        pltpu.SemaphoreType.DMA((2, nslot)),
        pltpu.SemaphoreType.DMA((nslot,)),
        pltpu.SemaphoreType.DMA((nslot,)),
    ]
    if with_counts:
        outs.append(jax.ShapeDtypeStruct((NC * NP_, CW), jnp.float32))
        scratch += [
            pltpu.VMEM((K, CW), jnp.float32),
            pltpu.VMEM_SHARED((NP_, CW), jnp.float32),
        ]

    def body(*refs):
        if with_counts:
            (table, eidx3, zrow, zc, ones_h, out, cnt_out,
             eidx, rows, acc_sh, isem, gsem, ssem, ones_v, cnt_sh) = refs
        else:
            (table, eidx3, zrow, out,
             eidx, rows, acc_sh, isem, gsem, ssem) = refs
        cid = lax.axis_index("c")
        sid = lax.axis_index("s")
        wid = sid * NC + cid
        r0 = sid * RPT
        c0 = wid * CHUNKS
        pltpu.sync_copy(zrow, acc_sh.at[pl.ds(r0, RPT)])
        if with_counts:
            pltpu.sync_copy(zc, cnt_sh.at[pl.ds(r0, RPT)])
            pltpu.sync_copy(ones_h, ones_v)
        plsc.subcore_barrier()

        def ifetch(b, s, i):
            pltpu.async_copy(eidx3.at[c0 + i], eidx.at[b, s], isem.at[b, s])

        def iwait(b, s, i):
            pltpu.make_async_copy(eidx3.at[c0 + i], eidx.at[b, s],
                                  isem.at[b, s]).wait()

        def gfetch(b, s):
            pltpu.async_copy(table.at[eidx.at[b, s, 0]], rows.at[s],
                             gsem.at[s])

        def gwait(b, s):
            pltpu.make_async_copy(table.at[eidx.at[b, s, 0]], rows.at[s],
                                  gsem.at[s]).wait()

        def sissue(b, s):
            pltpu.async_copy(rows.at[s], acc_sh.at[eidx.at[b, s, 1]],
                             ssem.at[s], add=True)
            if with_counts:
                pltpu.async_copy(ones_v, cnt_sh.at[eidx.at[b, s, 1]],
                                 ssem.at[s], add=True)

        def swait(b, s):
            pltpu.make_async_copy(rows.at[s], acc_sh.at[eidx.at[b, s, 1]],
                                  ssem.at[s]).wait()
            if with_counts:
                pltpu.make_async_copy(ones_v, cnt_sh.at[eidx.at[b, s, 1]],
                                      ssem.at[s]).wait()

        for s in range(nslot):
            ifetch(0, s, s)
        for s in range(nslot):
            if nslot + s < CHUNKS:
                ifetch(1, s, nslot + s)
        for s in range(nslot):
            iwait(0, s, s)
            gfetch(0, s)

        def do_group(g, bank):
            i0 = g * nslot
            for s in range(nslot):
                @pl.when(i0 + s < CHUNKS)
                def _(s=s):
                    gwait(bank, s)
                    sissue(bank, s)
            for s in range(nslot):
                j = i0 + nslot + s
                m = i0 + 2 * nslot + s

                @pl.when(i0 + s < CHUNKS)
                def _(s=s):
                    swait(bank, s)

                @pl.when(j < CHUNKS)
                def _(s=s, j=j):
                    iwait(1 - bank, s, j)
                    gfetch(1 - bank, s)

                @pl.when(m < CHUNKS)
                def _(s=s, m=m):
                    ifetch(bank, s, m)

            return None

        def step(p, carry):
            do_group(p * 2, 0)
            do_group(p * 2 + 1, 1)
            return carry

        lax.fori_loop(0, ngroups // 2, step, 0)
        plsc.subcore_barrier()
        o0 = cid * NP_ + r0
        pltpu.sync_copy(acc_sh.at[pl.ds(r0, RPT)], out.at[pl.ds(o0, RPT)])
        if with_counts:
            pltpu.sync_copy(cnt_sh.at[pl.ds(r0, RPT)], cnt_out.at[pl.ds(o0, RPT)])

    k = pl.kernel(body, out_type=tuple(outs), mesh=mesh,
                  scratch_types=scratch,
                  compiler_params=pltpu.CompilerParams(
                      use_tc_tiling_on_sc=False))
    _SC_AGG_CACHE[key] = k
    return k


BN = 1000
GRID = N // BN


def _tcb_body(a0, a1, cntT, x, w1l, w1r, b1, w2l, w2r, b2, g_out, hr_out):
    cnt = jnp.maximum(jnp.sum(cntT[...], axis=1, keepdims=True), 1.0)
    mean = (a0[...] + a1[...]) / cnt
    h = jnp.dot(mean, w1l[...], preferred_element_type=jnp.float32)
    h += jnp.dot(x[...], w1r[...], preferred_element_type=jnp.float32)
    h = jnp.maximum(h + b1[...], 0.0)
    g_out[...] = jnp.dot(h, w2l[...], preferred_element_type=jnp.float32)
    hr_out[...] = jnp.dot(h, w2r[...], preferred_element_type=jnp.float32) + b2[...]


_tcb = pl.pallas_call(
    _tcb_body,
    grid=(GRID,),
    in_specs=[
        pl.BlockSpec((BN, D), lambda i: (i, 0)),
        pl.BlockSpec((BN, D), lambda i: (i, 0)),
        pl.BlockSpec((BN, 2), lambda i: (i, 0)),
        pl.BlockSpec((BN, D), lambda i: (i, 0)),
        pl.BlockSpec((D, H), lambda i: (0, 0)),
        pl.BlockSpec((D, H), lambda i: (0, 0)),
        pl.BlockSpec((1, H), lambda i: (0, 0)),
        pl.BlockSpec((H, C), lambda i: (0, 0)),
        pl.BlockSpec((H, C), lambda i: (0, 0)),
        pl.BlockSpec((1, C), lambda i: (0, 0)),
    ],
    out_specs=[
        pl.BlockSpec((BN, C), lambda i: (i, 0)),
        pl.BlockSpec((BN, C), lambda i: (i, 0)),
    ],
    out_shape=[
        jax.ShapeDtypeStruct((N, C), jnp.float32),
        jax.ShapeDtypeStruct((N, C), jnp.float32),
    ],
)


def _tcc_body(a0, a1, cntT, hr, ra, rb, lw, lb, out):
    cnt = jnp.maximum(jnp.sum(cntT[...], axis=1, keepdims=True), 1.0)
    h2 = (a0[...] + a1[...]) / cnt + hr[...]
    nsq = jnp.sum(h2 * h2, axis=1, keepdims=True)
    norm_o = jnp.maximum(nsq, 1e-15)
    s1 = jnp.where(norm_o > MAXNORM, MAXNORM / norm_o, 1.0)
    p_norm = jnp.maximum(s1 * norm_o, 1e-15)
    z = jnp.clip(p_norm, -1.0 + 1e-7, 1.0 - 1e-7)
    artanh = 0.5 * jnp.log((1.0 + z) / (1.0 - z))
    alpha = s1 * artanh / p_norm
    a = jnp.dot(h2, ra[...], preferred_element_type=jnp.float32)
    b = jnp.dot(h2, rb[...], preferred_element_type=jnp.float32)
    acc = jnp.dot(a * b, lw[...], preferred_element_type=jnp.float32)
    h_euc = alpha * acc + lb[...]
    u_norm = jnp.maximum(
        jnp.sqrt(jnp.sum(h_euc * h_euc, axis=1, keepdims=True)), 1e-15)
    gamma = jnp.tanh(u_norm) * h_euc / u_norm
    n2 = jnp.maximum(
        jnp.sqrt(jnp.sum(gamma * gamma, axis=1, keepdims=True)), 1e-15)
    gamma = jnp.where(n2 > MAXNORM, gamma * (MAXNORM / n2), gamma)
    m = jnp.max(gamma, axis=1, keepdims=True)
    y = gamma - m
    out[...] = y - jnp.log(jnp.sum(jnp.exp(y), axis=1, keepdims=True))


_tcc = pl.pallas_call(
    _tcc_body,
    grid=(GRID,),
    in_specs=[
        pl.BlockSpec((BN, C), lambda i: (i, 0)),
        pl.BlockSpec((BN, C), lambda i: (i, 0)),
        pl.BlockSpec((BN, 2), lambda i: (i, 0)),
        pl.BlockSpec((BN, C), lambda i: (i, 0)),
        pl.BlockSpec((C, C * C), lambda i: (0, 0)),
        pl.BlockSpec((C, C * C), lambda i: (0, 0)),
        pl.BlockSpec((C * C, C), lambda i: (0, 0)),
        pl.BlockSpec((1, C), lambda i: (0, 0)),
    ],
    out_specs=pl.BlockSpec((BN, C), lambda i: (i, 0)),
    out_shape=jax.ShapeDtypeStruct((N, C), jnp.float32),
)


def kernel(x, edge_index, W1_l, W1_r, b1, W2_l, W2_r, b2, lin_W, lin_b):
    src = edge_index[0].astype(jnp.int32).reshape(NW, EPW)
    dst = edge_index[1].astype(jnp.int32).reshape(NW, EPW)
    pad_s = jnp.broadcast_to((jnp.arange(PAD, dtype=jnp.int32) * 89) % N,
                             (NW, PAD))
    pad_d = jnp.broadcast_to(N + (jnp.arange(PAD, dtype=jnp.int32) % (NP_ - N)),
                             (NW, PAD))
    src = jnp.concatenate([src, pad_s], axis=1).reshape(NW, CHUNKS, 1, K)
    dst = jnp.concatenate([dst, pad_d], axis=1).reshape(NW, CHUNKS, 1, K)
    eidx3 = jnp.concatenate([src, dst], axis=2).reshape(NW * CHUNKS, 2, K)
    zrow_d = jnp.zeros((RPT, D), jnp.float32)
    zrow_c = jnp.zeros((RPT, C), jnp.float32)
    zc = jnp.zeros((RPT, CW), jnp.float32)
    ones_h = jnp.ones((K, CW), jnp.float32)

    acc1, cnt = _make_sc_agg(D, True)(x, eidx3, zrow_d, zc, ones_h)
    cntT = jnp.concatenate([cnt[:N, 0:1], cnt[NP_:NP_ + N, 0:1]], axis=1)
    g, hr = _tcb(acc1[:N], acc1[NP_:NP_ + N], cntT, x, W1_l, W1_r,
                 b1.reshape(1, H), W2_l, W2_r, b2.reshape(1, C))
    (acc2,) = _make_sc_agg(C, False)(g, eidx3, zrow_c)
    ra = jnp.repeat(jnp.eye(C, dtype=jnp.float32), C, axis=1)
    rb = jnp.tile(jnp.eye(C, dtype=jnp.float32), (1, C))
    out = _tcc(acc2[:N], acc2[NP_:NP_ + N], cntT, hr,
               ra, rb, lin_W, lin_b.reshape(1, C))
    return out


# speedup vs baseline: 13.6345x; 1.0744x over previous
import jax
import jax.numpy as jnp
from jax import lax
from jax.experimental import pallas as pl
from jax.experimental.pallas import tpu as pltpu
from jax.experimental.pallas import tpu_sc as plsc

N = 10000
E = 320000
D = 128
H = 128
C = 16
MAXNORM = 1.0 - 4e-3

NC = 2
NS = 16
NW = NC * NS
EPW = E // NW
K = 128
CHUNKS = 79
PAD = CHUNKS * K - EPW
NP_ = 10240
RPT = NP_ // NS
CW = 16


_SC_AGG_CACHE = {}


def _make_sc_agg(width, with_counts):
    key = (width, with_counts)
    if key in _SC_AGG_CACHE:
        return _SC_AGG_CACHE[key]
    mesh = plsc.VectorSubcoreMesh(core_axis_name="c", subcore_axis_name="s",
                                  num_cores=NC, num_subcores=NS)
    nslot = 2 if width >= 128 else 4
    ngroups = -(-CHUNKS // nslot)
    assert ngroups % 2 == 0
    outs = [jax.ShapeDtypeStruct((NC * NP_, width), jnp.float32)]
    scratch = [
        pltpu.VMEM((2, nslot, 2, K), jnp.int32),
        pltpu.VMEM((nslot, K, width), jnp.float32),
        pltpu.VMEM_SHARED((NP_, width), jnp.float32),
        pltpu.SemaphoreType.DMA((2, nslot)),
        pltpu.SemaphoreType.DMA((nslot,)),
        pltpu.SemaphoreType.DMA((nslot,)),
    ]
    if with_counts:
        outs.append(jax.ShapeDtypeStruct((NC * NP_, CW), jnp.float32))
        scratch += [
            pltpu.VMEM((K, CW), jnp.float32),
            pltpu.VMEM_SHARED((NP_, CW), jnp.float32),
        ]

    def body(*refs):
        if with_counts:
            (table, eidx3, zrow, zc, ones_h, out, cnt_out,
             eidx, rows, acc_sh, isem, gsem, ssem, ones_v, cnt_sh) = refs
        else:
            (table, eidx3, zrow, out,
             eidx, rows, acc_sh, isem, gsem, ssem) = refs
        cid = lax.axis_index("c")
        sid = lax.axis_index("s")
        wid = sid * NC + cid
        r0 = sid * RPT
        c0 = wid * CHUNKS
        pltpu.sync_copy(zrow, acc_sh.at[pl.ds(r0, RPT)])
        if with_counts:
            pltpu.sync_copy(zc, cnt_sh.at[pl.ds(r0, RPT)])
            pltpu.sync_copy(ones_h, ones_v)
        plsc.subcore_barrier()

        def ifetch(b, s, i):
            pltpu.async_copy(eidx3.at[c0 + i], eidx.at[b, s], isem.at[b, s])

        def iwait(b, s, i):
            pltpu.make_async_copy(eidx3.at[c0 + i], eidx.at[b, s],
                                  isem.at[b, s]).wait()

        def gfetch(b, s):
            pltpu.async_copy(table.at[eidx.at[b, s, 0]], rows.at[s],
                             gsem.at[s])

        def gwait(b, s):
            pltpu.make_async_copy(table.at[eidx.at[b, s, 0]], rows.at[s],
                                  gsem.at[s]).wait()

        def sissue(b, s):
            pltpu.async_copy(rows.at[s], acc_sh.at[eidx.at[b, s, 1]],
                             ssem.at[s], add=True)
            if with_counts:
                pltpu.async_copy(ones_v, cnt_sh.at[eidx.at[b, s, 1]],
                                 ssem.at[s], add=True)

        def swait(b, s):
            pltpu.make_async_copy(rows.at[s], acc_sh.at[eidx.at[b, s, 1]],
                                  ssem.at[s]).wait()
            if with_counts:
                pltpu.make_async_copy(ones_v, cnt_sh.at[eidx.at[b, s, 1]],
                                      ssem.at[s]).wait()

        if nslot == 2:
            def buf(i):
                return ((i // 2) % 2, i % 2)

            ifetch(0, 0, 0)
            ifetch(0, 1, 1)
            ifetch(1, 0, 2)
            iwait(0, 0, 0)
            gfetch(0, 0)

            def rstep(p, carry):
                for o in range(4):
                    i = p * 4 + o
                    b, s = buf(o)
                    bp, sp = buf(o + 3)
                    bn, sn = buf(o + 1)

                    @pl.when(i < CHUNKS)
                    def _(b=b, s=s):
                        gwait(b, s)
                        sissue(b, s)

                    @pl.when((i >= 1) & (i - 1 < CHUNKS))
                    def _(bp=bp, sp=sp):
                        swait(bp, sp)

                    @pl.when(i + 1 < CHUNKS)
                    def _(bn=bn, sn=sn, i=i):
                        iwait(bn, sn, i + 1)
                        gfetch(bn, sn)

                    @pl.when(i + 3 < CHUNKS)
                    def _(bp=bp, sp=sp, i=i):
                        ifetch(bp, sp, i + 3)
                return carry

            lax.fori_loop(0, (CHUNKS + 4) // 4, rstep, 0)
            plsc.subcore_barrier()
            o0 = cid * NP_ + r0
            pltpu.sync_copy(acc_sh.at[pl.ds(r0, RPT)], out.at[pl.ds(o0, RPT)])
            if with_counts:
                pltpu.sync_copy(cnt_sh.at[pl.ds(r0, RPT)],
                                cnt_out.at[pl.ds(o0, RPT)])
            return

        for s in range(nslot):
            ifetch(0, s, s)
        for s in range(nslot):
            if nslot + s < CHUNKS:
                ifetch(1, s, nslot + s)
        for s in range(nslot):
            iwait(0, s, s)
            gfetch(0, s)

        def do_group(g, bank):
            i0 = g * nslot
            for s in range(nslot):
                @pl.when(i0 + s < CHUNKS)
                def _(s=s):
                    gwait(bank, s)
                    sissue(bank, s)
            for s in range(nslot):
                j = i0 + nslot + s
                m = i0 + 2 * nslot + s

                @pl.when(i0 + s < CHUNKS)
                def _(s=s):
                    swait(bank, s)

                @pl.when(j < CHUNKS)
                def _(s=s, j=j):
                    iwait(1 - bank, s, j)
                    gfetch(1 - bank, s)

                @pl.when(m < CHUNKS)
                def _(s=s, m=m):
                    ifetch(bank, s, m)

            return None

        def step(p, carry):
            do_group(p * 2, 0)
            do_group(p * 2 + 1, 1)
            return carry

        lax.fori_loop(0, ngroups // 2, step, 0)
        plsc.subcore_barrier()
        o0 = cid * NP_ + r0
        pltpu.sync_copy(acc_sh.at[pl.ds(r0, RPT)], out.at[pl.ds(o0, RPT)])
        if with_counts:
            pltpu.sync_copy(cnt_sh.at[pl.ds(r0, RPT)], cnt_out.at[pl.ds(o0, RPT)])

    k = pl.kernel(body, out_type=tuple(outs), mesh=mesh,
                  scratch_types=scratch,
                  compiler_params=pltpu.CompilerParams(
                      use_tc_tiling_on_sc=False))
    _SC_AGG_CACHE[key] = k
    return k


BN = 1000
GRID = N // BN


def _tcb_body(a0, a1, cntT, x, w1l, w1r, b1, w2l, w2r, b2, g_out, hr_out):
    cnt = jnp.maximum(jnp.sum(cntT[...], axis=1, keepdims=True), 1.0)
    mean = (a0[...] + a1[...]) / cnt
    h = jnp.dot(mean, w1l[...], preferred_element_type=jnp.float32)
    h += jnp.dot(x[...], w1r[...], preferred_element_type=jnp.float32)
    h = jnp.maximum(h + b1[...], 0.0)
    g_out[...] = jnp.dot(h, w2l[...], preferred_element_type=jnp.float32)
    hr_out[...] = jnp.dot(h, w2r[...], preferred_element_type=jnp.float32) + b2[...]


_tcb = pl.pallas_call(
    _tcb_body,
    grid=(GRID,),
    in_specs=[
        pl.BlockSpec((BN, D), lambda i: (i, 0)),
        pl.BlockSpec((BN, D), lambda i: (i, 0)),
        pl.BlockSpec((BN, 2), lambda i: (i, 0)),
        pl.BlockSpec((BN, D), lambda i: (i, 0)),
        pl.BlockSpec((D, H), lambda i: (0, 0)),
        pl.BlockSpec((D, H), lambda i: (0, 0)),
        pl.BlockSpec((1, H), lambda i: (0, 0)),
        pl.BlockSpec((H, C), lambda i: (0, 0)),
        pl.BlockSpec((H, C), lambda i: (0, 0)),
        pl.BlockSpec((1, C), lambda i: (0, 0)),
    ],
    out_specs=[
        pl.BlockSpec((BN, C), lambda i: (i, 0)),
        pl.BlockSpec((BN, C), lambda i: (i, 0)),
    ],
    out_shape=[
        jax.ShapeDtypeStruct((N, C), jnp.float32),
        jax.ShapeDtypeStruct((N, C), jnp.float32),
    ],
)


def _tcc_body(a0, a1, cntT, hr, ra, rb, lw, lb, out):
    cnt = jnp.maximum(jnp.sum(cntT[...], axis=1, keepdims=True), 1.0)
    h2 = (a0[...] + a1[...]) / cnt + hr[...]
    nsq = jnp.sum(h2 * h2, axis=1, keepdims=True)
    norm_o = jnp.maximum(nsq, 1e-15)
    s1 = jnp.where(norm_o > MAXNORM, MAXNORM / norm_o, 1.0)
    p_norm = jnp.maximum(s1 * norm_o, 1e-15)
    z = jnp.clip(p_norm, -1.0 + 1e-7, 1.0 - 1e-7)
    artanh = 0.5 * jnp.log((1.0 + z) / (1.0 - z))
    alpha = s1 * artanh / p_norm
    a = jnp.dot(h2, ra[...], preferred_element_type=jnp.float32)
    b = jnp.dot(h2, rb[...], preferred_element_type=jnp.float32)
    acc = jnp.dot(a * b, lw[...], preferred_element_type=jnp.float32)
    h_euc = alpha * acc + lb[...]
    u_norm = jnp.maximum(
        jnp.sqrt(jnp.sum(h_euc * h_euc, axis=1, keepdims=True)), 1e-15)
    gamma = jnp.tanh(u_norm) * h_euc / u_norm
    n2 = jnp.maximum(
        jnp.sqrt(jnp.sum(gamma * gamma, axis=1, keepdims=True)), 1e-15)
    gamma = jnp.where(n2 > MAXNORM, gamma * (MAXNORM / n2), gamma)
    m = jnp.max(gamma, axis=1, keepdims=True)
    y = gamma - m
    out[...] = y - jnp.log(jnp.sum(jnp.exp(y), axis=1, keepdims=True))


_tcc = pl.pallas_call(
    _tcc_body,
    grid=(GRID,),
    in_specs=[
        pl.BlockSpec((BN, C), lambda i: (i, 0)),
        pl.BlockSpec((BN, C), lambda i: (i, 0)),
        pl.BlockSpec((BN, 2), lambda i: (i, 0)),
        pl.BlockSpec((BN, C), lambda i: (i, 0)),
        pl.BlockSpec((C, C * C), lambda i: (0, 0)),
        pl.BlockSpec((C, C * C), lambda i: (0, 0)),
        pl.BlockSpec((C * C, C), lambda i: (0, 0)),
        pl.BlockSpec((1, C), lambda i: (0, 0)),
    ],
    out_specs=pl.BlockSpec((BN, C), lambda i: (i, 0)),
    out_shape=jax.ShapeDtypeStruct((N, C), jnp.float32),
)


def kernel(x, edge_index, W1_l, W1_r, b1, W2_l, W2_r, b2, lin_W, lin_b):
    src = edge_index[0].astype(jnp.int32).reshape(NW, EPW)
    dst = edge_index[1].astype(jnp.int32).reshape(NW, EPW)
    pad_s = jnp.broadcast_to((jnp.arange(PAD, dtype=jnp.int32) * 89) % N,
                             (NW, PAD))
    pad_d = jnp.broadcast_to(N + (jnp.arange(PAD, dtype=jnp.int32) % (NP_ - N)),
                             (NW, PAD))
    src = jnp.concatenate([src, pad_s], axis=1).reshape(NW, CHUNKS, 1, K)
    dst = jnp.concatenate([dst, pad_d], axis=1).reshape(NW, CHUNKS, 1, K)
    eidx3 = jnp.concatenate([src, dst], axis=2).reshape(NW * CHUNKS, 2, K)
    zrow_d = jnp.zeros((RPT, D), jnp.float32)
    zrow_c = jnp.zeros((RPT, C), jnp.float32)
    zc = jnp.zeros((RPT, CW), jnp.float32)
    ones_h = jnp.ones((K, CW), jnp.float32)

    acc1, cnt = _make_sc_agg(D, True)(x, eidx3, zrow_d, zc, ones_h)
    cntT = jnp.concatenate([cnt[:N, 0:1], cnt[NP_:NP_ + N, 0:1]], axis=1)
    g, hr = _tcb(acc1[:N], acc1[NP_:NP_ + N], cntT, x, W1_l, W1_r,
                 b1.reshape(1, H), W2_l, W2_r, b2.reshape(1, C))
    (acc2,) = _make_sc_agg(C, False)(g, eidx3, zrow_c)
    ra = jnp.repeat(jnp.eye(C, dtype=jnp.float32), C, axis=1)
    rb = jnp.tile(jnp.eye(C, dtype=jnp.float32), (1, C))
    out = _tcc(acc2[:N], acc2[NP_:NP_ + N], cntT, hr,
               ra, rb, lin_W, lin_b.reshape(1, C))
    return out


# speedup vs baseline: 14.5537x; 1.0674x over previous
import jax
import jax.numpy as jnp
from jax import lax
from jax.experimental import pallas as pl
from jax.experimental.pallas import tpu as pltpu
from jax.experimental.pallas import tpu_sc as plsc

N = 10000
E = 320000
D = 128
H = 128
C = 16
MAXNORM = 1.0 - 4e-3

NC = 2
NS = 16
NW = NC * NS
EPW = E // NW
K = 128
CHUNKS = 79
PAD = CHUNKS * K - EPW
NP_ = 10240
RPT = NP_ // NS
CW = 16


_SC_AGG_CACHE = {}


def _make_sc_agg(width, with_counts):
    key = (width, with_counts)
    if key in _SC_AGG_CACHE:
        return _SC_AGG_CACHE[key]
    mesh = plsc.VectorSubcoreMesh(core_axis_name="c", subcore_axis_name="s",
                                  num_cores=NC, num_subcores=NS)
    nslot = 2 if width >= 128 else 4
    ngroups = -(-CHUNKS // nslot)
    assert ngroups % 2 == 0
    outs = [jax.ShapeDtypeStruct((NC * NP_, width), jnp.float32)]
    scratch = [
        pltpu.VMEM((2, nslot, 2, K), jnp.int32),
        pltpu.VMEM((nslot, K, width), jnp.float32),
        pltpu.VMEM_SHARED((NP_, width), jnp.float32),
        pltpu.SemaphoreType.DMA((2, nslot)),
        pltpu.SemaphoreType.DMA((nslot,)),
        pltpu.SemaphoreType.DMA((nslot,)),
    ]
    if with_counts:
        outs.append(jax.ShapeDtypeStruct((NC * NP_, CW), jnp.float32))
        scratch += [
            pltpu.VMEM((K, CW), jnp.float32),
            pltpu.VMEM_SHARED((NP_, CW), jnp.float32),
        ]

    def body(*refs):
        if with_counts:
            (table, eidx3, zrow, zc, ones_h, out, cnt_out,
             eidx, rows, acc_sh, isem, gsem, ssem, ones_v, cnt_sh) = refs
        else:
            (table, eidx3, zrow, out,
             eidx, rows, acc_sh, isem, gsem, ssem) = refs
        cid = lax.axis_index("c")
        sid = lax.axis_index("s")
        wid = sid * NC + cid
        r0 = sid * RPT
        c0 = wid * CHUNKS
        pltpu.sync_copy(zrow, acc_sh.at[pl.ds(r0, RPT)])
        if with_counts:
            pltpu.sync_copy(zc, cnt_sh.at[pl.ds(r0, RPT)])
            pltpu.sync_copy(ones_h, ones_v)
        plsc.subcore_barrier()

        def ifetch(b, s, i):
            pltpu.async_copy(eidx3.at[c0 + i], eidx.at[b, s], isem.at[b, s])

        def iwait(b, s, i):
            pltpu.make_async_copy(eidx3.at[c0 + i], eidx.at[b, s],
                                  isem.at[b, s]).wait()

        def gfetch(b, s):
            pltpu.async_copy(table.at[eidx.at[b, s, 0]], rows.at[s],
                             gsem.at[s])

        def gwait(b, s):
            pltpu.make_async_copy(table.at[eidx.at[b, s, 0]], rows.at[s],
                                  gsem.at[s]).wait()

        def sissue(b, s):
            pltpu.async_copy(rows.at[s], acc_sh.at[eidx.at[b, s, 1]],
                             ssem.at[s], add=True)
            if with_counts:
                pltpu.async_copy(ones_v, cnt_sh.at[eidx.at[b, s, 1]],
                                 ssem.at[s], add=True)

        def swait(b, s):
            pltpu.make_async_copy(rows.at[s], acc_sh.at[eidx.at[b, s, 1]],
                                  ssem.at[s]).wait()
            if with_counts:
                pltpu.make_async_copy(ones_v, cnt_sh.at[eidx.at[b, s, 1]],
                                      ssem.at[s]).wait()

        if nslot == 2:
            def buf(i):
                return ((i // 2) % 2, i % 2)

            ifetch(0, 0, 0)
            ifetch(0, 1, 1)
            ifetch(1, 0, 2)
            iwait(0, 0, 0)
            gfetch(0, 0)

            def rstep(p, carry):
                for o in range(4):
                    i = p * 4 + o
                    b, s = buf(o)
                    bp, sp = buf(o + 3)
                    bn, sn = buf(o + 1)

                    @pl.when(i < CHUNKS)
                    def _(b=b, s=s):
                        gwait(b, s)
                        sissue(b, s)

                    @pl.when((i >= 1) & (i - 1 < CHUNKS))
                    def _(bp=bp, sp=sp):
                        swait(bp, sp)

                    @pl.when(i + 1 < CHUNKS)
                    def _(bn=bn, sn=sn, i=i):
                        iwait(bn, sn, i + 1)
                        gfetch(bn, sn)

                    @pl.when(i + 3 < CHUNKS)
                    def _(bp=bp, sp=sp, i=i):
                        ifetch(bp, sp, i + 3)
                return carry

            lax.fori_loop(0, (CHUNKS + 4) // 4, rstep, 0)
            plsc.subcore_barrier()
            o0 = cid * NP_ + r0
            pltpu.sync_copy(acc_sh.at[pl.ds(r0, RPT)], out.at[pl.ds(o0, RPT)])
            if with_counts:
                pltpu.sync_copy(cnt_sh.at[pl.ds(r0, RPT)],
                                cnt_out.at[pl.ds(o0, RPT)])
            return

        for s in range(nslot):
            ifetch(0, s, s)
        for s in range(nslot):
            if nslot + s < CHUNKS:
                ifetch(1, s, nslot + s)
        for s in range(nslot):
            iwait(0, s, s)
            gfetch(0, s)

        def do_group(g, bank):
            i0 = g * nslot
            for s in range(nslot):
                @pl.when(i0 + s < CHUNKS)
                def _(s=s):
                    gwait(bank, s)
                    sissue(bank, s)
            for s in range(nslot):
                j = i0 + nslot + s
                m = i0 + 2 * nslot + s

                @pl.when(i0 + s < CHUNKS)
                def _(s=s):
                    swait(bank, s)

                @pl.when(j < CHUNKS)
                def _(s=s, j=j):
                    iwait(1 - bank, s, j)
                    gfetch(1 - bank, s)

                @pl.when(m < CHUNKS)
                def _(s=s, m=m):
                    ifetch(bank, s, m)

            return None

        def step(p, carry):
            do_group(p * 2, 0)
            do_group(p * 2 + 1, 1)
            return carry

        lax.fori_loop(0, ngroups // 2, step, 0)
        plsc.subcore_barrier()
        o0 = cid * NP_ + r0
        pltpu.sync_copy(acc_sh.at[pl.ds(r0, RPT)], out.at[pl.ds(o0, RPT)])
        if with_counts:
            pltpu.sync_copy(cnt_sh.at[pl.ds(r0, RPT)], cnt_out.at[pl.ds(o0, RPT)])

    k = pl.kernel(body, out_type=tuple(outs), mesh=mesh,
                  scratch_types=scratch,
                  compiler_params=pltpu.CompilerParams(
                      use_tc_tiling_on_sc=False))
    _SC_AGG_CACHE[key] = k
    return k


BN = 1024
GRID = NP_ // BN
OFF = NP_ // BN


def _tcb_body(a0, a1, c0, c1, x, w1l, w1r, b1, w2l, w2r, b2, g_out, hr_out):
    cnt = jnp.maximum(c0[...][:, 0:1] + c1[...][:, 0:1], 1.0)
    mean = (a0[...] + a1[...]) / cnt
    h = jnp.dot(mean, w1l[...], preferred_element_type=jnp.float32)
    h += jnp.dot(x[...], w1r[...], preferred_element_type=jnp.float32)
    h = jnp.maximum(h + b1[...], 0.0)
    g_out[...] = jnp.dot(h, w2l[...], preferred_element_type=jnp.float32)
    hr_out[...] = jnp.dot(h, w2r[...], preferred_element_type=jnp.float32) + b2[...]


_tcb = pl.pallas_call(
    _tcb_body,
    grid=(GRID,),
    in_specs=[
        pl.BlockSpec((BN, D), lambda i: (i, 0)),
        pl.BlockSpec((BN, D), lambda i: (i + OFF, 0)),
        pl.BlockSpec((BN, CW), lambda i: (i, 0)),
        pl.BlockSpec((BN, CW), lambda i: (i + OFF, 0)),
        pl.BlockSpec((BN, D), lambda i: (i, 0)),
        pl.BlockSpec((D, H), lambda i: (0, 0)),
        pl.BlockSpec((D, H), lambda i: (0, 0)),
        pl.BlockSpec((1, H), lambda i: (0, 0)),
        pl.BlockSpec((H, C), lambda i: (0, 0)),
        pl.BlockSpec((H, C), lambda i: (0, 0)),
        pl.BlockSpec((1, C), lambda i: (0, 0)),
    ],
    out_specs=[
        pl.BlockSpec((BN, C), lambda i: (i, 0)),
        pl.BlockSpec((BN, C), lambda i: (i, 0)),
    ],
    out_shape=[
        jax.ShapeDtypeStruct((NP_, C), jnp.float32),
        jax.ShapeDtypeStruct((NP_, C), jnp.float32),
    ],
)


def _tcc_body(a0, a1, c0, c1, hr, ra, rb, lw, lb, out):
    cnt = jnp.maximum(c0[...][:, 0:1] + c1[...][:, 0:1], 1.0)
    h2 = (a0[...] + a1[...]) / cnt + hr[...]
    nsq = jnp.sum(h2 * h2, axis=1, keepdims=True)
    norm_o = jnp.maximum(nsq, 1e-15)
    s1 = jnp.where(norm_o > MAXNORM, MAXNORM / norm_o, 1.0)
    p_norm = jnp.maximum(s1 * norm_o, 1e-15)
    z = jnp.clip(p_norm, -1.0 + 1e-7, 1.0 - 1e-7)
    artanh = 0.5 * jnp.log((1.0 + z) / (1.0 - z))
    alpha = s1 * artanh / p_norm
    a = jnp.dot(h2, ra[...], preferred_element_type=jnp.float32)
    b = jnp.dot(h2, rb[...], preferred_element_type=jnp.float32)
    acc = jnp.dot(a * b, lw[...], preferred_element_type=jnp.float32)
    h_euc = alpha * acc + lb[...]
    u_norm = jnp.maximum(
        jnp.sqrt(jnp.sum(h_euc * h_euc, axis=1, keepdims=True)), 1e-15)
    gamma = jnp.tanh(u_norm) * h_euc / u_norm
    n2 = jnp.maximum(
        jnp.sqrt(jnp.sum(gamma * gamma, axis=1, keepdims=True)), 1e-15)
    gamma = jnp.where(n2 > MAXNORM, gamma * (MAXNORM / n2), gamma)
    m = jnp.max(gamma, axis=1, keepdims=True)
    y = gamma - m
    out[...] = y - jnp.log(jnp.sum(jnp.exp(y), axis=1, keepdims=True))


_tcc = pl.pallas_call(
    _tcc_body,
    grid=(GRID,),
    in_specs=[
        pl.BlockSpec((BN, C), lambda i: (i, 0)),
        pl.BlockSpec((BN, C), lambda i: (i + OFF, 0)),
        pl.BlockSpec((BN, CW), lambda i: (i, 0)),
        pl.BlockSpec((BN, CW), lambda i: (i + OFF, 0)),
        pl.BlockSpec((BN, C), lambda i: (i, 0)),
        pl.BlockSpec((C, C * C), lambda i: (0, 0)),
        pl.BlockSpec((C, C * C), lambda i: (0, 0)),
        pl.BlockSpec((C * C, C), lambda i: (0, 0)),
        pl.BlockSpec((1, C), lambda i: (0, 0)),
    ],
    out_specs=pl.BlockSpec((BN, C), lambda i: (i, 0)),
    out_shape=jax.ShapeDtypeStruct((NP_, C), jnp.float32),
)


def kernel(x, edge_index, W1_l, W1_r, b1, W2_l, W2_r, b2, lin_W, lin_b):
    src = edge_index[0].astype(jnp.int32).reshape(NW, EPW)
    dst = edge_index[1].astype(jnp.int32).reshape(NW, EPW)
    pad_s = jnp.broadcast_to((jnp.arange(PAD, dtype=jnp.int32) * 89) % N,
                             (NW, PAD))
    pad_d = jnp.broadcast_to(N + (jnp.arange(PAD, dtype=jnp.int32) % (NP_ - N)),
                             (NW, PAD))
    src = jnp.concatenate([src, pad_s], axis=1).reshape(NW, CHUNKS, 1, K)
    dst = jnp.concatenate([dst, pad_d], axis=1).reshape(NW, CHUNKS, 1, K)
    eidx3 = jnp.concatenate([src, dst], axis=2).reshape(NW * CHUNKS, 2, K)
    zrow_d = jnp.zeros((RPT, D), jnp.float32)
    zrow_c = jnp.zeros((RPT, C), jnp.float32)
    zc = jnp.zeros((RPT, CW), jnp.float32)
    ones_h = jnp.ones((K, CW), jnp.float32)

    acc1, cnt = _make_sc_agg(D, True)(x, eidx3, zrow_d, zc, ones_h)
    g, hr = _tcb(acc1, acc1, cnt, cnt, x, W1_l, W1_r,
                 b1.reshape(1, H), W2_l, W2_r, b2.reshape(1, C))
    (acc2,) = _make_sc_agg(C, False)(g, eidx3, zrow_c)
    ra = jnp.repeat(jnp.eye(C, dtype=jnp.float32), C, axis=1)
    rb = jnp.tile(jnp.eye(C, dtype=jnp.float32), (1, C))
    out = _tcc(acc2, acc2, cnt, cnt, hr,
               ra, rb, lin_W, lin_b.reshape(1, C))
    return out[:N]
